# baseline (speedup 1.0000x reference)
import functools
from math import pi as PI

import jax
import jax.numpy as jnp
from jax import lax
from jax.experimental import pallas as pl
from jax.experimental.pallas import tpu as pltpu
from jax.experimental.pallas import tpu_sc as plsc

N = 10000
E = 320000
H = 128
NG = 50
CUTOFF = 10.0
EPS = 1e-5

NC = 2
NS = 16
NW = NC * NS
CH = 128
NHALF = 5
EH = E // NHALF
NCHUNK = EH // CH
NP = 10240

_mesh = plsc.VectorSubcoreMesh(core_axis_name="c", subcore_axis_name="s")


NCH = EH // CH
CEIL = (NCH + NW - 1) // NW


@functools.partial(
    pl.kernel,
    out_type=(jax.ShapeDtypeStruct((EH, H), jnp.float32),
              jax.ShapeDtypeStruct((EH, H), jnp.float32)),
    mesh=_mesh,
    scratch_types=[
        pltpu.VMEM((CEIL, CH), jnp.int32),
        pltpu.VMEM((CEIL, CH), jnp.int32),
        pltpu.VMEM((2, CH, H), jnp.float32),
        pltpu.VMEM((2, CH, H), jnp.float32),
        pltpu.SemaphoreType.DMA,
        pltpu.SemaphoreType.DMA,
        pltpu.SemaphoreType.DMA,
        pltpu.SemaphoreType.DMA,
    ],
)
def _sc_gather(v_hbm, ii3_hbm, jj3_hbm, vi_out, vj_out,
               idx_i, idx_j, rows_i, rows_j,
               sem_g0, sem_g1, sem_w0, sem_w1):
    wid = lax.axis_index("s") * NC + lax.axis_index("c")
    ci = pltpu.async_copy(ii3_hbm.at[:, wid, :], idx_i, sem_g0)
    cj = pltpu.async_copy(jj3_hbm.at[:, wid, :], idx_j, sem_g1)
    ci.wait()
    cj.wait()

    nloop = (NCH - wid + NW - 1) // NW
    sem_g = (sem_g0, sem_g1)
    sem_w = (sem_w0, sem_w1)

    def issue_gather(t, b):
        pltpu.async_copy(v_hbm.at[idx_i.at[t]], rows_i.at[b], sem_g[b])
        pltpu.async_copy(v_hbm.at[idx_j.at[t]], rows_j.at[b], sem_g[b])

    def wait_gather(b):
        pltpu.make_async_copy(v_hbm.at[idx_i.at[0]], rows_i.at[b],
                              sem_g[b]).wait()
        pltpu.make_async_copy(v_hbm.at[idx_j.at[0]], rows_j.at[b],
                              sem_g[b]).wait()

    def issue_wb(t, b):
        base = (wid + t * NW) * CH
        pltpu.async_copy(rows_i.at[b], vi_out.at[pl.ds(base, CH)], sem_w[b])
        pltpu.async_copy(rows_j.at[b], vj_out.at[pl.ds(base, CH)], sem_w[b])

    def wait_wb(t, b):
        base = (wid + t * NW) * CH
        pltpu.make_async_copy(rows_i.at[b], vi_out.at[pl.ds(base, CH)],
                              sem_w[b]).wait()
        pltpu.make_async_copy(rows_j.at[b], vj_out.at[pl.ds(base, CH)],
                              sem_w[b]).wait()

    issue_gather(0, 0)
    issue_gather(1, 1)

    def pair(p, carry):
        for b in range(2):
            t = 2 * p + b

            @pl.when(t < nloop)
            def _():
                wait_gather(b)
                issue_wb(t, b)
                nt = t + 2

                @pl.when(nt < nloop)
                def _():
                    wait_wb(t, b)
                    issue_gather(nt, b)
        return carry

    lax.fori_loop(0, (CEIL + 1) // 2, pair, 0)
    wait_wb(0, 0)
    wait_wb(0, 1)


@functools.partial(
    pl.kernel,
    out_type=(jax.ShapeDtypeStruct((NP, H), jnp.float32),
              jax.ShapeDtypeStruct((NP, H), jnp.float32)),
    mesh=_mesh,
    scratch_types=[
        pltpu.VMEM((CEIL, CH), jnp.int32),
        pltpu.VMEM((2, CH, H), jnp.float32),
        pltpu.VMEM_SHARED((NP, H), jnp.float32),
        pltpu.SemaphoreType.DMA,
        pltpu.SemaphoreType.DMA,
        pltpu.SemaphoreType.DMA,
        pltpu.SemaphoreType.DMA,
    ],
)
def _sc_scatter(msg_hbm, ii3_hbm, init0_hbm, init1_hbm, out0, out1,
                idx2, rows, acc_sh, sem_l0, sem_l1, sem_s0, sem_s1):
    c = lax.axis_index("c")
    s = lax.axis_index("s")
    wid = s * NC + c
    rpt = NP // NS
    pltpu.async_copy(ii3_hbm.at[:, wid, :], idx2, sem_l0)
    @pl.when(c == 0)
    def _():
        pltpu.sync_copy(init0_hbm.at[pl.ds(s * rpt, rpt)],
                        acc_sh.at[pl.ds(s * rpt, rpt)])

    @pl.when(c == 1)
    def _():
        pltpu.sync_copy(init1_hbm.at[pl.ds(s * rpt, rpt)],
                        acc_sh.at[pl.ds(s * rpt, rpt)])

    pltpu.make_async_copy(ii3_hbm.at[:, wid, :], idx2, sem_l0).wait()
    plsc.subcore_barrier()

    nloop = (NCH - wid + NW - 1) // NW
    sem_l = (sem_l0, sem_l1)
    sem_s = (sem_s0, sem_s1)

    def issue_load(t, b):
        pltpu.async_copy(msg_hbm.at[pl.ds((wid + t * NW) * CH, CH)],
                         rows.at[b], sem_l[b])

    def wait_load(t, b):
        pltpu.make_async_copy(msg_hbm.at[pl.ds((wid + t * NW) * CH, CH)],
                              rows.at[b], sem_l[b]).wait()

    def issue_scat(t, b):
        pltpu.async_copy(rows.at[b], acc_sh.at[idx2.at[t]], sem_s[b],
                         add=True)

    def wait_scat(t, b):
        pltpu.make_async_copy(rows.at[b], acc_sh.at[idx2.at[t]],
                              sem_s[b]).wait()

    issue_load(0, 0)
    issue_load(1, 1)

    def pair(p, carry):
        for b in range(2):
            t = 2 * p + b

            @pl.when(t < nloop)
            def _():
                wait_load(t, b)
                issue_scat(t, b)
                nt = t + 2

                @pl.when(nt < nloop)
                def _():
                    wait_scat(t, b)
                    issue_load(nt, b)
        return carry

    lax.fori_loop(0, CEIL // 2, pair, 0)
    wait_scat(0, 0)
    wait_scat(0, 1)
    plsc.subcore_barrier()

    @pl.when(c == 0)
    def _():
        pltpu.sync_copy(acc_sh.at[pl.ds(s * rpt, rpt)], out0.at[pl.ds(s * rpt, rpt)])

    @pl.when(c == 1)
    def _():
        pltpu.sync_copy(acc_sh.at[pl.ds(s * rpt, rpt)], out1.at[pl.ds(s * rpt, rpt)])


BE = 3200
GRIDH = EH // BE

_LOG2E = 1.4426950408889634
_LN2 = 0.6931471805599453


def _softplus(x):
    return jnp.log2(1.0 + jnp.exp2(x * _LOG2E)) * _LN2


def _sigmoid(x):
    return 1.0 / (1.0 + jnp.exp2(x * (-_LOG2E)))


def _mm_body(vi_ref, vj_ref, de_ref, wvv_ref, wd_ref, b_ref,
             x_ref, st_ref, acc_ref):
    e = pl.program_id(0)
    a = jnp.concatenate([vi_ref[...], vj_ref[...]], axis=1).astype(jnp.bfloat16)
    x = jnp.dot(a, wvv_ref[...], preferred_element_type=jnp.float32)
    x = x + lax.dot_general(de_ref[...].astype(jnp.bfloat16), wd_ref[...],
                            (((0,), (0,)), ((), ())),
                            preferred_element_type=jnp.float32)
    x = x + b_ref[...]
    x_ref[...] = x.astype(jnp.bfloat16)

    @pl.when(e == 0)
    def _():
        acc_ref[...] = jnp.zeros_like(acc_ref)

    acc_ref[...] += jnp.concatenate(
        [jnp.sum(x, 0, keepdims=True), jnp.sum(x * x, 0, keepdims=True)], 0)

    @pl.when(e == GRIDH - 1)
    def _():
        st_ref[...] = acc_ref[...]


def _make_mm(half):
    off = half * GRIDH
    return pl.pallas_call(
        _mm_body,
        grid=(GRIDH,),
        in_specs=[
            pl.BlockSpec((BE, H), lambda e: (e, 0)),
            pl.BlockSpec((BE, H), lambda e: (e, 0)),
            pl.BlockSpec((NG, BE), lambda e: (0, e + off)),
            pl.BlockSpec((2 * H, 2 * H), lambda e: (0, 0)),
            pl.BlockSpec((NG, 2 * H), lambda e: (0, 0)),
            pl.BlockSpec((1, 2 * H), lambda e: (0, 0)),
        ],
        out_specs=[
            pl.BlockSpec((BE, 2 * H), lambda e: (e, 0)),
            pl.BlockSpec((2, 2 * H), lambda e: (0, 0)),
        ],
        out_shape=[
            jax.ShapeDtypeStruct((EH, 2 * H), jnp.bfloat16),
            jax.ShapeDtypeStruct((2, 2 * H), jnp.float32),
        ],
        scratch_shapes=[pltpu.VMEM((2, 2 * H), jnp.float32)],
    )


_mm = [_make_mm(h) for h in range(NHALF)]


def _act_body(x_ref, st_ref, g_ref, b_ref, d_ref, msg_ref):
    st = st_ref[...]
    mean = st[0:1, :] * (1.0 / E)
    var = st[1:2, :] * (1.0 / E) - mean * mean
    xn = ((x_ref[...].astype(jnp.float32) - mean) * lax.rsqrt(var + EPS)
          * g_ref[...] + b_ref[...])
    cpart = xn[:, :H]
    fpart = xn[:, H:]
    m = _softplus(cpart) * _sigmoid(fpart)
    cf = 0.5 * (jnp.cos(d_ref[...].reshape(1, BE) * (PI / CUTOFF)) + 1.0)
    msg_ref[...] = m * jnp.transpose(cf, (1, 0))


def _make_act(half):
    off = half * GRIDH
    return pl.pallas_call(
        _act_body,
        grid=(GRIDH,),
        in_specs=[
            pl.BlockSpec((BE, 2 * H), lambda e: (e, 0)),
            pl.BlockSpec((2, 2 * H), lambda e: (0, 0)),
            pl.BlockSpec((1, 2 * H), lambda e: (0, 0)),
            pl.BlockSpec((1, 2 * H), lambda e: (0, 0)),
            pl.BlockSpec((1, 1, BE), lambda e: (e + off, 0, 0)),
        ],
        out_specs=pl.BlockSpec((BE, H), lambda e: (e, 0)),
        out_shape=jax.ShapeDtypeStruct((EH, H), jnp.float32),
    )


_act = [_make_act(h) for h in range(NHALF)]


def _fin_body(p0_ref, p1_ref, v_ref, g2_ref, b2_ref,
              gw1_ref, gw2_ref, gb_ref, out_ref):
    x = p0_ref[:N, :] + p1_ref[:N, :]
    mean = jnp.mean(x, 0, keepdims=True)
    var = jnp.mean(x * x, 0, keepdims=True) - mean * mean
    xn = (x - mean) * lax.rsqrt(var + EPS) * g2_ref[...] + b2_ref[...]
    s = _sigmoid(
        jnp.dot(v_ref[...], gw1_ref[...], preferred_element_type=jnp.float32)
        + jnp.dot(xn, gw2_ref[...], preferred_element_type=jnp.float32)
        + gb_ref[...])
    out_ref[...] = _softplus(s * v_ref[...] + (1.0 - s) * xn)


_fin = pl.pallas_call(
    _fin_body,
    out_shape=jax.ShapeDtypeStruct((N, H), jnp.float32),
)


def kernel(v, dist, dist_emb, edge_index, Wsf_w, Wsf_b,
           bn1_g, bn1_b, bn2_g, bn2_b, gru_w, gru_b):
    jj = edge_index[0].astype(jnp.int32)
    ii = edge_index[1].astype(jnp.int32)
    wvv = Wsf_w[:, :2 * H].T.astype(jnp.bfloat16)
    wd = Wsf_w[:, 2 * H:].T.astype(jnp.bfloat16)
    bias = Wsf_b.reshape(1, 2 * H)
    dembT = jnp.transpose(dist_emb, (1, 0))
    dist3 = dist.reshape(E // BE, 1, BE)
    zeros = jnp.zeros((NP, H), jnp.float32)

    def _mk3(a):
        pad = jnp.zeros((CEIL * NW * CH - EH,), jnp.int32)
        return jnp.concatenate([a, pad]).reshape(CEIL, NW, CH)

    ii3h = [_mk3(ii[h * EH:(h + 1) * EH]) for h in range(NHALF)]
    jj3h = [_mk3(jj[h * EH:(h + 1) * EH]) for h in range(NHALF)]

    gath = [_sc_gather(v, ii3h[h], jj3h[h]) for h in range(NHALF)]
    mm = [_mm[h](gath[h][0], gath[h][1], dembT, wvv, wd, bias)
          for h in range(NHALF)]
    st = mm[0][1]
    for h in range(1, NHALF):
        st = st + mm[h][1]
    p0, p1 = zeros, zeros
    for h in range(NHALF):
        msg = _act[h](mm[h][0], st, bn1_g.reshape(1, 2 * H),
                      bn1_b.reshape(1, 2 * H), dist3)
        p0, p1 = _sc_scatter(msg, ii3h[h], p0, p1)
    out = _fin(p0, p1, v,
               bn2_g.reshape(1, H), bn2_b.reshape(1, H),
               gru_w[:, :H].T, gru_w[:, H:].T, gru_b.reshape(1, H))
    return out


# speedup vs baseline: 1.0003x; 1.0003x over previous
import functools
from math import pi as PI

import jax
import jax.numpy as jnp
from jax import lax
from jax.experimental import pallas as pl
from jax.experimental.pallas import tpu as pltpu
from jax.experimental.pallas import tpu_sc as plsc

N = 10000
E = 320000
H = 128
NG = 50
CUTOFF = 10.0
EPS = 1e-5

NC = 2
NS = 16
NW = NC * NS
CH = 128
NHALF = 4
EH = E // NHALF
NCHUNK = EH // CH
NP = 10240

_mesh = plsc.VectorSubcoreMesh(core_axis_name="c", subcore_axis_name="s")


NCH = EH // CH
CEIL = (NCH + NW - 1) // NW


@functools.partial(
    pl.kernel,
    out_type=(jax.ShapeDtypeStruct((EH, H), jnp.float32),
              jax.ShapeDtypeStruct((EH, H), jnp.float32)),
    mesh=_mesh,
    scratch_types=[
        pltpu.VMEM((CEIL, CH), jnp.int32),
        pltpu.VMEM((CEIL, CH), jnp.int32),
        pltpu.VMEM((3, CH, H), jnp.float32),
        pltpu.VMEM((3, CH, H), jnp.float32),
        pltpu.SemaphoreType.DMA,
        pltpu.SemaphoreType.DMA,
        pltpu.SemaphoreType.DMA,
        pltpu.SemaphoreType.DMA,
        pltpu.SemaphoreType.DMA,
        pltpu.SemaphoreType.DMA,
    ],
)
def _sc_gather(v_hbm, ii3_hbm, jj3_hbm, vi_out, vj_out,
               idx_i, idx_j, rows_i, rows_j,
               sem_g0, sem_g1, sem_g2, sem_w0, sem_w1, sem_w2):
    wid = lax.axis_index("s") * NC + lax.axis_index("c")
    ci = pltpu.async_copy(ii3_hbm.at[:, wid, :], idx_i, sem_g0)
    cj = pltpu.async_copy(jj3_hbm.at[:, wid, :], idx_j, sem_g1)
    ci.wait()
    cj.wait()

    nloop = (NCH - wid + NW - 1) // NW
    sem_g = (sem_g0, sem_g1, sem_g2)
    sem_w = (sem_w0, sem_w1, sem_w2)

    def issue_gather(t, b):
        pltpu.async_copy(v_hbm.at[idx_i.at[t]], rows_i.at[b], sem_g[b])
        pltpu.async_copy(v_hbm.at[idx_j.at[t]], rows_j.at[b], sem_g[b])

    def wait_gather(b):
        pltpu.make_async_copy(v_hbm.at[idx_i.at[0]], rows_i.at[b],
                              sem_g[b]).wait()
        pltpu.make_async_copy(v_hbm.at[idx_j.at[0]], rows_j.at[b],
                              sem_g[b]).wait()

    def issue_wb(t, b):
        base = (wid + t * NW) * CH
        pltpu.async_copy(rows_i.at[b], vi_out.at[pl.ds(base, CH)], sem_w[b])
        pltpu.async_copy(rows_j.at[b], vj_out.at[pl.ds(base, CH)], sem_w[b])

    def wait_wb(t, b):
        base = (wid + t * NW) * CH
        pltpu.make_async_copy(rows_i.at[b], vi_out.at[pl.ds(base, CH)],
                              sem_w[b]).wait()
        pltpu.make_async_copy(rows_j.at[b], vj_out.at[pl.ds(base, CH)],
                              sem_w[b]).wait()

    issue_gather(0, 0)
    issue_gather(1, 1)
    issue_gather(2, 2)

    def trip(p, carry):
        for b in range(3):
            t = 3 * p + b

            @pl.when(t < nloop)
            def _():
                wait_gather(b)
                issue_wb(t, b)
                nt = t + 3

                @pl.when(nt < nloop)
                def _():
                    wait_wb(t, b)
                    issue_gather(nt, b)
        return carry

    lax.fori_loop(0, (CEIL + 2) // 3, trip, 0)
    wait_wb(0, 0)
    wait_wb(0, 1)
    wait_wb(0, 2)


@functools.partial(
    pl.kernel,
    out_type=(jax.ShapeDtypeStruct((NP, H), jnp.float32),
              jax.ShapeDtypeStruct((NP, H), jnp.float32)),
    mesh=_mesh,
    scratch_types=[
        pltpu.VMEM((CEIL, CH), jnp.int32),
        pltpu.VMEM((2, CH, H), jnp.float32),
        pltpu.VMEM_SHARED((NP, H), jnp.float32),
        pltpu.SemaphoreType.DMA,
        pltpu.SemaphoreType.DMA,
        pltpu.SemaphoreType.DMA,
        pltpu.SemaphoreType.DMA,
    ],
)
def _sc_scatter(msg_hbm, ii3_hbm, init0_hbm, init1_hbm, out0, out1,
                idx2, rows, acc_sh, sem_l0, sem_l1,
                sem_s0, sem_s1):
    c = lax.axis_index("c")
    s = lax.axis_index("s")
    wid = s * NC + c
    rpt = NP // NS
    pltpu.async_copy(ii3_hbm.at[:, wid, :], idx2, sem_l0)
    @pl.when(c == 0)
    def _():
        pltpu.sync_copy(init0_hbm.at[pl.ds(s * rpt, rpt)],
                        acc_sh.at[pl.ds(s * rpt, rpt)])

    @pl.when(c == 1)
    def _():
        pltpu.sync_copy(init1_hbm.at[pl.ds(s * rpt, rpt)],
                        acc_sh.at[pl.ds(s * rpt, rpt)])

    pltpu.make_async_copy(ii3_hbm.at[:, wid, :], idx2, sem_l0).wait()
    plsc.subcore_barrier()

    nloop = (NCH - wid + NW - 1) // NW
    sem_l = (sem_l0, sem_l1)
    sem_s = (sem_s0, sem_s1)

    def issue_load(t, b):
        pltpu.async_copy(msg_hbm.at[pl.ds((wid + t * NW) * CH, CH)],
                         rows.at[b], sem_l[b])

    def wait_load(t, b):
        pltpu.make_async_copy(msg_hbm.at[pl.ds((wid + t * NW) * CH, CH)],
                              rows.at[b], sem_l[b]).wait()

    def issue_scat(t, b):
        pltpu.async_copy(rows.at[b], acc_sh.at[idx2.at[t]], sem_s[b],
                         add=True)

    def wait_scat(t, b):
        pltpu.make_async_copy(rows.at[b], acc_sh.at[idx2.at[t]],
                              sem_s[b]).wait()

    issue_load(0, 0)
    issue_load(1, 1)

    def pair(p, carry):
        for b in range(2):
            t = 2 * p + b

            @pl.when(t < nloop)
            def _():
                wait_load(t, b)
                issue_scat(t, b)
                nt = t + 2

                @pl.when(nt < nloop)
                def _():
                    wait_scat(t, b)
                    issue_load(nt, b)
        return carry

    lax.fori_loop(0, (CEIL + 1) // 2, pair, 0)
    wait_scat(0, 0)
    wait_scat(0, 1)
    plsc.subcore_barrier()

    @pl.when(c == 0)
    def _():
        pltpu.sync_copy(acc_sh.at[pl.ds(s * rpt, rpt)], out0.at[pl.ds(s * rpt, rpt)])

    @pl.when(c == 1)
    def _():
        pltpu.sync_copy(acc_sh.at[pl.ds(s * rpt, rpt)], out1.at[pl.ds(s * rpt, rpt)])


BE = 3200
GRIDH = EH // BE

_LOG2E = 1.4426950408889634
_LN2 = 0.6931471805599453


def _softplus(x):
    return jnp.log2(1.0 + jnp.exp2(x * _LOG2E)) * _LN2


def _sigmoid(x):
    return 1.0 / (1.0 + jnp.exp2(x * (-_LOG2E)))


def _mm_body(vi_ref, vj_ref, de_ref, wvv_ref, wd_ref, b_ref,
             x_ref, st_ref, acc_ref):
    e = pl.program_id(0)
    a = jnp.concatenate([vi_ref[...], vj_ref[...]], axis=1).astype(jnp.bfloat16)
    x = jnp.dot(a, wvv_ref[...], preferred_element_type=jnp.float32)
    x = x + lax.dot_general(de_ref[...].astype(jnp.bfloat16), wd_ref[...],
                            (((0,), (0,)), ((), ())),
                            preferred_element_type=jnp.float32)
    x = x + b_ref[...]
    x_ref[...] = x.astype(jnp.bfloat16)

    @pl.when(e == 0)
    def _():
        acc_ref[...] = jnp.zeros_like(acc_ref)

    acc_ref[...] += jnp.concatenate(
        [jnp.sum(x, 0, keepdims=True), jnp.sum(x * x, 0, keepdims=True)], 0)

    @pl.when(e == GRIDH - 1)
    def _():
        st_ref[...] = acc_ref[...]


def _make_mm(half):
    off = half * GRIDH
    return pl.pallas_call(
        _mm_body,
        grid=(GRIDH,),
        in_specs=[
            pl.BlockSpec((BE, H), lambda e: (e, 0)),
            pl.BlockSpec((BE, H), lambda e: (e, 0)),
            pl.BlockSpec((NG, BE), lambda e: (0, e + off)),
            pl.BlockSpec((2 * H, 2 * H), lambda e: (0, 0)),
            pl.BlockSpec((NG, 2 * H), lambda e: (0, 0)),
            pl.BlockSpec((1, 2 * H), lambda e: (0, 0)),
        ],
        out_specs=[
            pl.BlockSpec((BE, 2 * H), lambda e: (e, 0)),
            pl.BlockSpec((2, 2 * H), lambda e: (0, 0)),
        ],
        out_shape=[
            jax.ShapeDtypeStruct((EH, 2 * H), jnp.bfloat16),
            jax.ShapeDtypeStruct((2, 2 * H), jnp.float32),
        ],
        scratch_shapes=[pltpu.VMEM((2, 2 * H), jnp.float32)],
    )


_mm = [_make_mm(h) for h in range(NHALF)]


def _act_body(x_ref, st_ref, g_ref, b_ref, d_ref, msg_ref):
    st = st_ref[...]
    mean = st[0:1, :] * (1.0 / E)
    var = st[1:2, :] * (1.0 / E) - mean * mean
    xn = ((x_ref[...].astype(jnp.float32) - mean) * lax.rsqrt(var + EPS)
          * g_ref[...] + b_ref[...])
    cpart = xn[:, :H]
    fpart = xn[:, H:]
    m = _softplus(cpart) * _sigmoid(fpart)
    cf = 0.5 * (jnp.cos(d_ref[...].reshape(1, BE) * (PI / CUTOFF)) + 1.0)
    msg_ref[...] = m * jnp.transpose(cf, (1, 0))


def _make_act(half):
    off = half * GRIDH
    return pl.pallas_call(
        _act_body,
        grid=(GRIDH,),
        in_specs=[
            pl.BlockSpec((BE, 2 * H), lambda e: (e, 0)),
            pl.BlockSpec((2, 2 * H), lambda e: (0, 0)),
            pl.BlockSpec((1, 2 * H), lambda e: (0, 0)),
            pl.BlockSpec((1, 2 * H), lambda e: (0, 0)),
            pl.BlockSpec((1, 1, BE), lambda e: (e + off, 0, 0)),
        ],
        out_specs=pl.BlockSpec((BE, H), lambda e: (e, 0)),
        out_shape=jax.ShapeDtypeStruct((EH, H), jnp.float32),
    )


_act = [_make_act(h) for h in range(NHALF)]


def _fin_body(p0_ref, p1_ref, v_ref, g2_ref, b2_ref,
              gw1_ref, gw2_ref, gb_ref, out_ref):
    x = p0_ref[:N, :] + p1_ref[:N, :]
    mean = jnp.mean(x, 0, keepdims=True)
    var = jnp.mean(x * x, 0, keepdims=True) - mean * mean
    xn = (x - mean) * lax.rsqrt(var + EPS) * g2_ref[...] + b2_ref[...]
    s = _sigmoid(
        jnp.dot(v_ref[...], gw1_ref[...], preferred_element_type=jnp.float32)
        + jnp.dot(xn, gw2_ref[...], preferred_element_type=jnp.float32)
        + gb_ref[...])
    out_ref[...] = _softplus(s * v_ref[...] + (1.0 - s) * xn)


_fin = pl.pallas_call(
    _fin_body,
    out_shape=jax.ShapeDtypeStruct((N, H), jnp.float32),
)


def kernel(v, dist, dist_emb, edge_index, Wsf_w, Wsf_b,
           bn1_g, bn1_b, bn2_g, bn2_b, gru_w, gru_b):
    jj = edge_index[0].astype(jnp.int32)
    ii = edge_index[1].astype(jnp.int32)
    wvv = Wsf_w[:, :2 * H].T.astype(jnp.bfloat16)
    wd = Wsf_w[:, 2 * H:].T.astype(jnp.bfloat16)
    bias = Wsf_b.reshape(1, 2 * H)
    dembT = jnp.transpose(dist_emb, (1, 0))
    dist3 = dist.reshape(E // BE, 1, BE)
    zeros = jnp.zeros((NP, H), jnp.float32)

    def _mk3(a):
        pad = jnp.zeros((CEIL * NW * CH - EH,), jnp.int32)
        return jnp.concatenate([a, pad]).reshape(CEIL, NW, CH)

    ii3h = [_mk3(ii[h * EH:(h + 1) * EH]) for h in range(NHALF)]
    jj3h = [_mk3(jj[h * EH:(h + 1) * EH]) for h in range(NHALF)]

    gath = [_sc_gather(v, ii3h[h], jj3h[h]) for h in range(NHALF)]
    mm = [_mm[h](gath[h][0], gath[h][1], dembT, wvv, wd, bias)
          for h in range(NHALF)]
    st = mm[0][1]
    for h in range(1, NHALF):
        st = st + mm[h][1]
    p0, p1 = zeros, zeros
    for h in range(NHALF):
        msg = _act[h](mm[h][0], st, bn1_g.reshape(1, 2 * H),
                      bn1_b.reshape(1, 2 * H), dist3)
        p0, p1 = _sc_scatter(msg, ii3h[h], p0, p1)
    out = _fin(p0, p1, v,
               bn2_g.reshape(1, H), bn2_b.reshape(1, H),
               gru_w[:, :H].T, gru_w[:, H:].T, gru_b.reshape(1, H))
    return out
